# Initial kernel scaffold; baseline (speedup 1.0000x reference)
import jax
import jax.numpy as jnp
from jax.experimental import pallas as pl


def kernel(node_features, edge_index, edge_features, batch, params):
    raise NotImplementedError("write your pallas kernel here")


# speedup vs baseline: 1.0303x; 1.0303x over previous
import functools

import jax
import jax.numpy as jnp
from jax import lax
from jax.experimental import pallas as pl
from jax.experimental.pallas import tpu as pltpu
from jax.experimental.pallas import tpu_sc as plsc

NC = 2
NS = 16
NW = NC * NS
CKS = 64
CKG = 64

NBLK = 400
EBLK = 2000


def _linear_kernel(x_ref, w_ref, b_ref, o_ref):
    o_ref[...] = jnp.dot(x_ref[...], w_ref[...],
                         preferred_element_type=jnp.float32) + b_ref[...]


def _tc_linear(x, w, b, blk=NBLK):
    n, k = x.shape
    m = w.shape[1]
    return pl.pallas_call(
        _linear_kernel,
        grid=(n // blk,),
        in_specs=[
            pl.BlockSpec((blk, k), lambda i: (i, 0)),
            pl.BlockSpec((k, m), lambda i: (0, 0)),
            pl.BlockSpec((1, m), lambda i: (0, 0)),
        ],
        out_specs=pl.BlockSpec((blk, m), lambda i: (i, 0)),
        out_shape=jax.ShapeDtypeStruct((n, m), jnp.float32),
    )(x, w, b)


def _ab_kernel(x_ref, wr_ref, wc_ref, a_ref, b_ref):
    x = x_ref[...]
    a_ref[...] = jnp.dot(x, wr_ref[...], preferred_element_type=jnp.float32)
    b_ref[...] = jnp.dot(x, wc_ref[...], preferred_element_type=jnp.float32)


def _tc_ab(x, wr, wc):
    n, h = x.shape
    return pl.pallas_call(
        _ab_kernel,
        grid=(n // NBLK,),
        in_specs=[
            pl.BlockSpec((NBLK, h), lambda i: (i, 0)),
            pl.BlockSpec((h, h), lambda i: (0, 0)),
            pl.BlockSpec((h, h), lambda i: (0, 0)),
        ],
        out_specs=[
            pl.BlockSpec((NBLK, h), lambda i: (i, 0)),
            pl.BlockSpec((NBLK, h), lambda i: (i, 0)),
        ],
        out_shape=[
            jax.ShapeDtypeStruct((n, h), jnp.float32),
            jax.ShapeDtypeStruct((n, h), jnp.float32),
        ],
    )(x, wr, wc)


def _edge_kernel(g1_ref, g2_ref, ef_ref, m_ref, cb_ref, v_ref, r_ref, lg_ref):
    pre = (g1_ref[...] + g2_ref[...]
           + jnp.dot(ef_ref[...], m_ref[...],
                     preferred_element_type=jnp.float32)
           + cb_ref[...])
    r = jnp.maximum(pre, 0.0)
    r_ref[...] = r
    logit = lax.dot_general(v_ref[...], r, (((1,), (1,)), ((), ())),
                            preferred_element_type=jnp.float32)
    lg_ref[...] = logit.reshape(1, 1, EBLK)


def _tc_edge(g1, g2, efp, m, cb, v):
    e = g1.shape[0]
    nb = e // EBLK
    return pl.pallas_call(
        _edge_kernel,
        grid=(nb,),
        in_specs=[
            pl.BlockSpec((EBLK, 128), lambda i: (i, 0)),
            pl.BlockSpec((EBLK, 128), lambda i: (i, 0)),
            pl.BlockSpec((EBLK, 16), lambda i: (i, 0)),
            pl.BlockSpec((16, 128), lambda i: (0, 0)),
            pl.BlockSpec((1, 128), lambda i: (0, 0)),
            pl.BlockSpec((1, 128), lambda i: (0, 0)),
        ],
        out_specs=[
            pl.BlockSpec((EBLK, 128), lambda i: (i, 0)),
            pl.BlockSpec((1, 1, EBLK), lambda i: (i, 0, 0)),
        ],
        out_shape=[
            jax.ShapeDtypeStruct((e, 128), jnp.float32),
            jax.ShapeDtypeStruct((nb, 1, EBLK), jnp.float32),
        ],
    )(g1, g2, efp, m, cb, v)


def _softmax_stats_kernel(lg_ref, o_ref, m_ref, z_ref):
    i = pl.program_id(0)

    @pl.when(i == 0)
    def _init():
        m_ref[0, 0] = -jnp.inf
        z_ref[0, 0] = 0.0

    blk = lg_ref[0, 0, :]
    bm = jnp.max(blk)
    m_old = m_ref[0, 0]
    m_new = jnp.maximum(m_old, bm)
    z_new = (z_ref[0, 0] * jnp.exp(m_old - m_new)
             + jnp.sum(jnp.exp(blk - m_new)))
    m_ref[0, 0] = m_new
    z_ref[0, 0] = z_new
    lane = lax.broadcasted_iota(jnp.int32, (1, 128), 1)
    o_ref[...] = jnp.where(lane == 0, m_new,
                           jnp.where(lane == 1, z_new, 0.0))


def _tc_softmax_stats(lg):
    nb = lg.shape[0]
    return pl.pallas_call(
        _softmax_stats_kernel,
        grid=(nb,),
        in_specs=[pl.BlockSpec((1, 1, EBLK), lambda i: (i, 0, 0))],
        out_specs=pl.BlockSpec((1, 128), lambda i: (0, 0)),
        out_shape=jax.ShapeDtypeStruct((1, 128), jnp.float32),
        scratch_shapes=[pltpu.SMEM((1, 1), jnp.float32),
                        pltpu.SMEM((1, 1), jnp.float32)],
    )(lg)


def _pscale_kernel(r_ref, lg_ref, st_ref, p_ref, w_ref):
    m = st_ref[0, 0]
    z = st_ref[0, 1]
    w = jnp.exp(lg_ref[0] - m) * (1.0 / z)
    wmat = lax.dot_general(w, jnp.ones((1, 128), jnp.float32),
                           (((0,), (0,)), ((), ())),
                           preferred_element_type=jnp.float32)
    p_ref[...] = r_ref[...] * wmat
    w_ref[...] = w.reshape(1, 1, EBLK)


def _tc_pscale(r, lg, stats):
    e = r.shape[0]
    nb = e // EBLK
    return pl.pallas_call(
        _pscale_kernel,
        grid=(nb,),
        in_specs=[
            pl.BlockSpec((EBLK, 128), lambda i: (i, 0)),
            pl.BlockSpec((1, 1, EBLK), lambda i: (i, 0, 0)),
            pl.BlockSpec((1, 128), lambda i: (0, 0)),
        ],
        out_specs=[
            pl.BlockSpec((EBLK, 128), lambda i: (i, 0)),
            pl.BlockSpec((1, 1, EBLK), lambda i: (i, 0, 0)),
        ],
        out_shape=[
            jax.ShapeDtypeStruct((e, 128), jnp.float32),
            jax.ShapeDtypeStruct((nb, 1, EBLK), jnp.float32),
        ],
    )(r, lg, stats)


def _upd_kernel(x_ref, s_ref, sp_ref, w2_ref, b2_ref, wa_ref, wb_ref,
                bu1_ref, wu2_ref, bu2_ref, o_ref):
    x = x_ref[...]
    s2d = sp_ref[0, 0:1, :] + sp_ref[0, 1:2, :]
    smat = lax.dot_general(s2d, jnp.ones((1, 128), jnp.float32),
                           (((0,), (0,)), ((), ())),
                           preferred_element_type=jnp.float32)
    agg = jnp.dot(s_ref[...], w2_ref[...], preferred_element_type=jnp.float32)
    agg = agg + smat * b2_ref[...]
    h = jnp.maximum(
        jnp.dot(x, wa_ref[...], preferred_element_type=jnp.float32)
        + jnp.dot(agg, wb_ref[...], preferred_element_type=jnp.float32)
        + bu1_ref[...], 0.0)
    upd = jnp.dot(h, wu2_ref[...], preferred_element_type=jnp.float32) \
        + bu2_ref[...]
    o_ref[...] = x + upd


def _tc_update(x, s, spart3, w2, b2, wa, wb, bu1, wu2, bu2):
    n = x.shape[0]
    return pl.pallas_call(
        _upd_kernel,
        grid=(n // NBLK,),
        in_specs=[
            pl.BlockSpec((NBLK, 128), lambda i: (i, 0)),
            pl.BlockSpec((NBLK, 128), lambda i: (i, 0)),
            pl.BlockSpec((1, 2, NBLK), lambda i: (i, 0, 0)),
            pl.BlockSpec((128, 128), lambda i: (0, 0)),
            pl.BlockSpec((1, 128), lambda i: (0, 0)),
            pl.BlockSpec((128, 128), lambda i: (0, 0)),
            pl.BlockSpec((128, 128), lambda i: (0, 0)),
            pl.BlockSpec((1, 128), lambda i: (0, 0)),
            pl.BlockSpec((128, 128), lambda i: (0, 0)),
            pl.BlockSpec((1, 128), lambda i: (0, 0)),
        ],
        out_specs=pl.BlockSpec((NBLK, 128), lambda i: (i, 0)),
        out_shape=jax.ShapeDtypeStruct((n, 128), jnp.float32),
    )(x, s, spart3, w2, b2, wa, wb, bu1, wu2, bu2)


def _readout_kernel(x_ref, bt_ref, wr_ref, br_ref, o_ref,
                    sum_ref, max_ref, cnt_ref):
    i = pl.program_id(0)
    nb = pl.num_programs(0)

    @pl.when(i == 0)
    def _init():
        sum_ref[...] = jnp.zeros((64, 128), jnp.float32)
        max_ref[...] = jnp.full((64, 128), -jnp.inf, jnp.float32)
        cnt_ref[...] = jnp.zeros((64, 128), jnp.float32)

    x = x_ref[...]
    b = bt_ref[...]
    oh = (b[:, 0:64] == lax.broadcasted_iota(jnp.int32, (NBLK, 64), 1)
          ).astype(jnp.float32)
    sum_ref[...] += lax.dot_general(oh, x, (((0,), (0,)), ((), ())),
                                    preferred_element_type=jnp.float32)
    cnt_ref[...] += lax.dot_general(oh, jnp.ones((NBLK, 128), jnp.float32),
                                    (((0,), (0,)), ((), ())),
                                    preferred_element_type=jnp.float32)

    blo = jnp.min(b)
    bhi = jnp.max(b)
    seg = lax.broadcasted_iota(jnp.int32, (64, 128), 0)

    def mbody(bb, _):
        mb = jnp.max(jnp.where(b == bb, x, -jnp.inf), axis=0, keepdims=True)
        max_ref[...] = jnp.maximum(
            max_ref[...], jnp.where(seg == bb, mb, -jnp.inf))
        return 0

    lax.fori_loop(blo, bhi + 1, mbody, 0)

    @pl.when(i == nb - 1)
    def _final():
        mean = sum_ref[...] / jnp.maximum(cnt_ref[...], 1.0)
        g = jnp.concatenate([mean, max_ref[...]], axis=1)
        o_ref[...] = jnp.maximum(
            jnp.dot(g, wr_ref[...], preferred_element_type=jnp.float32)
            + br_ref[...], 0.0)


def _tc_readout(x, batchb, wr, br):
    n = x.shape[0]
    return pl.pallas_call(
        _readout_kernel,
        grid=(n // NBLK,),
        in_specs=[
            pl.BlockSpec((NBLK, 128), lambda i: (i, 0)),
            pl.BlockSpec((NBLK, 128), lambda i: (i, 0)),
            pl.BlockSpec((256, 128), lambda i: (0, 0)),
            pl.BlockSpec((1, 128), lambda i: (0, 0)),
        ],
        out_specs=pl.BlockSpec((64, 128), lambda i: (0, 0)),
        out_shape=jax.ShapeDtypeStruct((64, 128), jnp.float32),
        scratch_shapes=[pltpu.VMEM((64, 128), jnp.float32),
                        pltpu.VMEM((64, 128), jnp.float32),
                        pltpu.VMEM((64, 128), jnp.float32)],
    )(x, batchb, wr, br)


def _sc_gather(a, b, row, col):
    n = a.shape[0]
    e = row.shape[0]
    nchunks = e // CKG
    mesh = plsc.VectorSubcoreMesh(core_axis_name="c", subcore_axis_name="s")

    @functools.partial(
        pl.kernel, mesh=mesh,
        out_type=(jax.ShapeDtypeStruct((e, 128), jnp.float32),
                  jax.ShapeDtypeStruct((e, 128), jnp.float32)),
        scratch_types=[
            pltpu.VMEM((CKG,), jnp.int32),
            pltpu.VMEM((CKG,), jnp.int32),
            pltpu.VMEM((CKG, 128), jnp.float32),
            pltpu.VMEM((CKG, 128), jnp.float32),
            pltpu.SemaphoreType.DMA,
            pltpu.SemaphoreType.DMA,
        ],
    )
    def k(a_hbm, b_hbm, row_hbm, col_hbm, g1_hbm, g2_hbm,
          idx1, idx2, buf1, buf2, sem1, sem2):
        wid = lax.axis_index("s") * NC + lax.axis_index("c")

        def body(i, _):
            base = (wid + i * NW) * CKG
            pltpu.sync_copy(row_hbm.at[pl.ds(base, CKG)], idx1)
            pltpu.sync_copy(col_hbm.at[pl.ds(base, CKG)], idx2)
            cp1 = pltpu.async_copy(a_hbm.at[idx1], buf1, sem1)
            cp2 = pltpu.async_copy(b_hbm.at[idx2], buf2, sem2)
            cp1.wait()
            cp2.wait()
            pltpu.sync_copy(buf1, g1_hbm.at[pl.ds(base, CKG)])
            pltpu.sync_copy(buf2, g2_hbm.at[pl.ds(base, CKG)])
            return 0

        nmine = (nchunks - wid + NW - 1) // NW
        lax.fori_loop(0, nmine, body, 0)

    return k(a, b, row, col)


def _sc_scatter(p, w1, col, z128, z1):
    npad = z128.shape[0] * 4
    e = col.shape[0]
    ncnk = npad // 4
    nchunks = e // CKS
    rpt = ncnk // NS
    spt = npad // NS
    mesh = plsc.VectorSubcoreMesh(core_axis_name="c", subcore_axis_name="s")

    @functools.partial(
        pl.kernel, mesh=mesh,
        out_type=(jax.ShapeDtypeStruct((npad, 128), jnp.float32),
                  jax.ShapeDtypeStruct((2, npad), jnp.float32)),
        scratch_types=[
            pltpu.VMEM_SHARED((ncnk + 8, 128), jnp.float32),
            pltpu.VMEM_SHARED((npad,), jnp.float32),
            pltpu.VMEM((CKS,), jnp.int32),
            pltpu.VMEM((CKS,), jnp.int32),
            pltpu.VMEM((CKS, 128), jnp.float32),
            pltpu.VMEM((CKS,), jnp.float32),
        ],
    )
    def k(p_hbm, w_hbm, col_hbm, z128_hbm, z1_hbm, s_hbm, sp_hbm,
          acc, sacc, ibuf, libuf, pbuf, wbuf):
        cid = lax.axis_index("c")
        sid = lax.axis_index("s")
        wid = sid * NC + cid

        for pp in range(2):
            nbase = (cid * 2 + pp) * ncnk
            pltpu.sync_copy(z128_hbm.at[pl.ds(sid * rpt, rpt)],
                            acc.at[pl.ds(sid * rpt, rpt)])

            @pl.when(sid == 0)
            def _zdump():
                pltpu.sync_copy(z128_hbm.at[pl.ds(0, 8)],
                                acc.at[pl.ds(ncnk, 8)])

            plsc.subcore_barrier()

            def body(i, _):
                base = (sid + i * NS) * CKS
                pltpu.sync_copy(col_hbm.at[pl.ds(base, CKS)], ibuf)
                pltpu.sync_copy(p_hbm.at[pl.ds(base, CKS)], pbuf)
                for kk in range(CKS // 16):
                    v = ibuf[pl.ds(kk * 16, 16)] - nbase
                    ok = (v >= 0) & (v < ncnk)
                    libuf[pl.ds(kk * 16, 16)] = jnp.where(ok, v, ncnk)
                pltpu.sync_copy(pbuf, acc.at[libuf], add=True)
                return 0

            nmine = (nchunks - sid + NS - 1) // NS
            lax.fori_loop(0, nmine, body, 0)
            plsc.subcore_barrier()
            pltpu.sync_copy(acc.at[pl.ds(sid * rpt, rpt)],
                            s_hbm.at[pl.ds(nbase + sid * rpt, rpt)])
            plsc.subcore_barrier()

        pltpu.sync_copy(z1_hbm.at[pl.ds(sid * spt, spt)],
                        sacc.at[pl.ds(sid * spt, spt)])
        plsc.subcore_barrier()

        def sbody(i, _):
            base = (wid + i * NW) * CKS
            pltpu.sync_copy(col_hbm.at[pl.ds(base, CKS)], ibuf)
            pltpu.sync_copy(w_hbm.at[pl.ds(base, CKS)], wbuf)
            pltpu.sync_copy(wbuf, sacc.at[ibuf], add=True)
            return 0

        nmine2 = (nchunks - wid + NW - 1) // NW
        lax.fori_loop(0, nmine2, sbody, 0)
        plsc.subcore_barrier()
        pltpu.sync_copy(sacc.at[pl.ds(sid * spt, spt)],
                        sp_hbm.at[cid, pl.ds(sid * spt, spt)])

    return k(p, w1, col, z128, z1)


def kernel(node_features, edge_index, edge_features, batch, params):
    n, nd = node_features.shape
    e = edge_index.shape[1]
    p = params
    h = 128
    nlayers = p['gW1'].shape[0]

    row = edge_index[0]
    col = edge_index[1]

    nfp = jnp.pad(node_features, ((0, 0), (0, h - nd)))
    wnp = jnp.pad(p['Wn'], ((0, h - nd), (0, 0)))
    efp = jnp.pad(edge_features, ((0, 0), (0, 16 - edge_features.shape[1])))

    w1r = p['gW1'][:, :h, :]
    w1c = p['gW1'][:, h:2 * h, :]
    w1e = p['gW1'][:, 2 * h:, :]
    m_l = jnp.pad(jnp.einsum('eh,lhk->lek', p['We'], w1e),
                  ((0, 0), (0, 6), (0, 0)))
    cb_l = (jnp.einsum('h,lhk->lk', p['be'], w1e)
            + p['gb1']).reshape(nlayers, 1, h)
    v_l = jnp.einsum('lhk,lko->lho', p['gW2'], p['gWa'])[..., 0] \
        .reshape(nlayers, 1, h)
    b2 = p['gb2'].reshape(nlayers, 1, h)
    wu1a = p['gWu1'][:, :h, :]
    wu1b = p['gWu1'][:, h:, :]
    bu1 = p['gbu1'].reshape(nlayers, 1, h)
    bu2 = p['gbu2'].reshape(nlayers, 1, h)
    br = p['br'].reshape(1, h)

    npad = 51200
    z128 = jnp.zeros((npad // 4, 128), jnp.float32)
    z1 = jnp.zeros((npad,), jnp.float32)
    batchb = jnp.broadcast_to(batch[:, None], (n, 128))

    x = _tc_linear(nfp, wnp, p['bn'].reshape(1, h))

    for i in range(nlayers):
        a, b = _tc_ab(x, w1r[i], w1c[i])
        g1, g2 = _sc_gather(a, b, row, col)
        r, lg = _tc_edge(g1, g2, efp, m_l[i], cb_l[i], v_l[i])
        stats = _tc_softmax_stats(lg)
        pm, w3 = _tc_pscale(r, lg, stats)
        w1 = w3.reshape(e)
        s, spart = _sc_scatter(pm, w1, col, z128, z1)
        spart3 = spart.reshape(2, npad // NBLK, NBLK).transpose(1, 0, 2)
        x = _tc_update(x, s, spart3, p['gW2'][i], b2[i],
                       wu1a[i], wu1b[i], bu1[i], p['gWu2'][i], bu2[i])

    return _tc_readout(x, batchb, p['Wr'].reshape(256, h), br)


# speedup vs baseline: 1.1693x; 1.1349x over previous
import functools

import jax
import jax.numpy as jnp
from jax import lax
from jax.experimental import pallas as pl
from jax.experimental.pallas import tpu as pltpu
from jax.experimental.pallas import tpu_sc as plsc

NC = 2
NS = 16
NW = NC * NS
CKS = 32
CKG = 32

NBLK = 400
EBLK = 2000


def _linear_kernel(x_ref, w_ref, b_ref, o_ref):
    o_ref[...] = jnp.dot(x_ref[...], w_ref[...],
                         preferred_element_type=jnp.float32) + b_ref[...]


def _tc_linear(x, w, b, blk=NBLK):
    n, k = x.shape
    m = w.shape[1]
    return pl.pallas_call(
        _linear_kernel,
        grid=(n // blk,),
        in_specs=[
            pl.BlockSpec((blk, k), lambda i: (i, 0)),
            pl.BlockSpec((k, m), lambda i: (0, 0)),
            pl.BlockSpec((1, m), lambda i: (0, 0)),
        ],
        out_specs=pl.BlockSpec((blk, m), lambda i: (i, 0)),
        out_shape=jax.ShapeDtypeStruct((n, m), jnp.float32),
    )(x, w, b)


def _ab_kernel(x_ref, wr_ref, wc_ref, a_ref, b_ref):
    x = x_ref[...]
    a_ref[...] = jnp.dot(x, wr_ref[...], preferred_element_type=jnp.float32)
    b_ref[...] = jnp.dot(x, wc_ref[...], preferred_element_type=jnp.float32)


def _tc_ab(x, wr, wc):
    n, h = x.shape
    return pl.pallas_call(
        _ab_kernel,
        grid=(n // NBLK,),
        in_specs=[
            pl.BlockSpec((NBLK, h), lambda i: (i, 0)),
            pl.BlockSpec((h, h), lambda i: (0, 0)),
            pl.BlockSpec((h, h), lambda i: (0, 0)),
        ],
        out_specs=[
            pl.BlockSpec((NBLK, h), lambda i: (i, 0)),
            pl.BlockSpec((NBLK, h), lambda i: (i, 0)),
        ],
        out_shape=[
            jax.ShapeDtypeStruct((n, h), jnp.float32),
            jax.ShapeDtypeStruct((n, h), jnp.float32),
        ],
    )(x, wr, wc)


def _edge_kernel(g1_ref, g2_ref, ef_ref, m_ref, cb_ref, v_ref, r_ref, lg_ref):
    pre = (g1_ref[...] + g2_ref[...]
           + jnp.dot(ef_ref[...], m_ref[...],
                     preferred_element_type=jnp.float32)
           + cb_ref[...])
    r = jnp.maximum(pre, 0.0)
    r_ref[...] = r
    logit = lax.dot_general(v_ref[...], r, (((1,), (1,)), ((), ())),
                            preferred_element_type=jnp.float32)
    lg_ref[...] = logit.reshape(1, 1, EBLK)


def _tc_edge(g1, g2, efp, m, cb, v):
    e = g1.shape[0]
    nb = e // EBLK
    return pl.pallas_call(
        _edge_kernel,
        grid=(nb,),
        in_specs=[
            pl.BlockSpec((EBLK, 128), lambda i: (i, 0)),
            pl.BlockSpec((EBLK, 128), lambda i: (i, 0)),
            pl.BlockSpec((EBLK, 16), lambda i: (i, 0)),
            pl.BlockSpec((16, 128), lambda i: (0, 0)),
            pl.BlockSpec((1, 128), lambda i: (0, 0)),
            pl.BlockSpec((1, 128), lambda i: (0, 0)),
        ],
        out_specs=[
            pl.BlockSpec((EBLK, 128), lambda i: (i, 0)),
            pl.BlockSpec((1, 1, EBLK), lambda i: (i, 0, 0)),
        ],
        out_shape=[
            jax.ShapeDtypeStruct((e, 128), jnp.float32),
            jax.ShapeDtypeStruct((nb, 1, EBLK), jnp.float32),
        ],
    )(g1, g2, efp, m, cb, v)


def _softmax_stats_kernel(lg_ref, o_ref, m_ref, z_ref):
    i = pl.program_id(0)

    @pl.when(i == 0)
    def _init():
        m_ref[0, 0] = -jnp.inf
        z_ref[0, 0] = 0.0

    blk = lg_ref[0, 0, :]
    bm = jnp.max(blk)
    m_old = m_ref[0, 0]
    m_new = jnp.maximum(m_old, bm)
    z_new = (z_ref[0, 0] * jnp.exp(m_old - m_new)
             + jnp.sum(jnp.exp(blk - m_new)))
    m_ref[0, 0] = m_new
    z_ref[0, 0] = z_new
    lane = lax.broadcasted_iota(jnp.int32, (1, 128), 1)
    o_ref[...] = jnp.where(lane == 0, m_new,
                           jnp.where(lane == 1, z_new, 0.0))


def _tc_softmax_stats(lg):
    nb = lg.shape[0]
    return pl.pallas_call(
        _softmax_stats_kernel,
        grid=(nb,),
        in_specs=[pl.BlockSpec((1, 1, EBLK), lambda i: (i, 0, 0))],
        out_specs=pl.BlockSpec((1, 128), lambda i: (0, 0)),
        out_shape=jax.ShapeDtypeStruct((1, 128), jnp.float32),
        scratch_shapes=[pltpu.SMEM((1, 1), jnp.float32),
                        pltpu.SMEM((1, 1), jnp.float32)],
    )(lg)


def _pscale_kernel(r_ref, lg_ref, st_ref, p_ref, w_ref):
    m = st_ref[0, 0]
    z = st_ref[0, 1]
    w = jnp.exp(lg_ref[0] - m) * (1.0 / z)
    wmat = lax.dot_general(w, jnp.ones((1, 128), jnp.float32),
                           (((0,), (0,)), ((), ())),
                           preferred_element_type=jnp.float32)
    p_ref[...] = r_ref[...] * wmat
    w_ref[...] = w.reshape(1, 1, EBLK)


def _tc_pscale(r, lg, stats):
    e = r.shape[0]
    nb = e // EBLK
    return pl.pallas_call(
        _pscale_kernel,
        grid=(nb,),
        in_specs=[
            pl.BlockSpec((EBLK, 128), lambda i: (i, 0)),
            pl.BlockSpec((1, 1, EBLK), lambda i: (i, 0, 0)),
            pl.BlockSpec((1, 128), lambda i: (0, 0)),
        ],
        out_specs=[
            pl.BlockSpec((EBLK, 128), lambda i: (i, 0)),
            pl.BlockSpec((1, 1, EBLK), lambda i: (i, 0, 0)),
        ],
        out_shape=[
            jax.ShapeDtypeStruct((e, 128), jnp.float32),
            jax.ShapeDtypeStruct((nb, 1, EBLK), jnp.float32),
        ],
    )(r, lg, stats)


def _upd_kernel(x_ref, s_ref, sp_ref, w2_ref, b2_ref, wa_ref, wb_ref,
                bu1_ref, wu2_ref, bu2_ref, o_ref):
    x = x_ref[...]
    s2d = sp_ref[0, 0:1, :] + sp_ref[0, 1:2, :]
    smat = lax.dot_general(s2d, jnp.ones((1, 128), jnp.float32),
                           (((0,), (0,)), ((), ())),
                           preferred_element_type=jnp.float32)
    agg = jnp.dot(s_ref[...], w2_ref[...], preferred_element_type=jnp.float32)
    agg = agg + smat * b2_ref[...]
    h = jnp.maximum(
        jnp.dot(x, wa_ref[...], preferred_element_type=jnp.float32)
        + jnp.dot(agg, wb_ref[...], preferred_element_type=jnp.float32)
        + bu1_ref[...], 0.0)
    upd = jnp.dot(h, wu2_ref[...], preferred_element_type=jnp.float32) \
        + bu2_ref[...]
    o_ref[...] = x + upd


def _tc_update(x, s, spart3, w2, b2, wa, wb, bu1, wu2, bu2):
    n = x.shape[0]
    return pl.pallas_call(
        _upd_kernel,
        grid=(n // NBLK,),
        in_specs=[
            pl.BlockSpec((NBLK, 128), lambda i: (i, 0)),
            pl.BlockSpec((NBLK, 128), lambda i: (i, 0)),
            pl.BlockSpec((1, 2, NBLK), lambda i: (i, 0, 0)),
            pl.BlockSpec((128, 128), lambda i: (0, 0)),
            pl.BlockSpec((1, 128), lambda i: (0, 0)),
            pl.BlockSpec((128, 128), lambda i: (0, 0)),
            pl.BlockSpec((128, 128), lambda i: (0, 0)),
            pl.BlockSpec((1, 128), lambda i: (0, 0)),
            pl.BlockSpec((128, 128), lambda i: (0, 0)),
            pl.BlockSpec((1, 128), lambda i: (0, 0)),
        ],
        out_specs=pl.BlockSpec((NBLK, 128), lambda i: (i, 0)),
        out_shape=jax.ShapeDtypeStruct((n, 128), jnp.float32),
    )(x, s, spart3, w2, b2, wa, wb, bu1, wu2, bu2)


def _readout_kernel(x_ref, bt_ref, wr_ref, br_ref, o_ref,
                    sum_ref, max_ref, cnt_ref):
    i = pl.program_id(0)
    nb = pl.num_programs(0)

    @pl.when(i == 0)
    def _init():
        sum_ref[...] = jnp.zeros((64, 128), jnp.float32)
        max_ref[...] = jnp.full((64, 128), -jnp.inf, jnp.float32)
        cnt_ref[...] = jnp.zeros((64, 128), jnp.float32)

    x = x_ref[...]
    b = bt_ref[...]
    oh = (b[:, 0:64] == lax.broadcasted_iota(jnp.int32, (NBLK, 64), 1)
          ).astype(jnp.float32)
    sum_ref[...] += lax.dot_general(oh, x, (((0,), (0,)), ((), ())),
                                    preferred_element_type=jnp.float32)
    cnt_ref[...] += lax.dot_general(oh, jnp.ones((NBLK, 128), jnp.float32),
                                    (((0,), (0,)), ((), ())),
                                    preferred_element_type=jnp.float32)

    blo = jnp.min(b)
    bhi = jnp.max(b)
    seg = lax.broadcasted_iota(jnp.int32, (64, 128), 0)

    def mbody(bb, _):
        mb = jnp.max(jnp.where(b == bb, x, -jnp.inf), axis=0, keepdims=True)
        max_ref[...] = jnp.maximum(
            max_ref[...], jnp.where(seg == bb, mb, -jnp.inf))
        return 0

    lax.fori_loop(blo, bhi + 1, mbody, 0)

    @pl.when(i == nb - 1)
    def _final():
        mean = sum_ref[...] / jnp.maximum(cnt_ref[...], 1.0)
        g = jnp.concatenate([mean, max_ref[...]], axis=1)
        o_ref[...] = jnp.maximum(
            jnp.dot(g, wr_ref[...], preferred_element_type=jnp.float32)
            + br_ref[...], 0.0)


def _tc_readout(x, batchb, wr, br):
    n = x.shape[0]
    return pl.pallas_call(
        _readout_kernel,
        grid=(n // NBLK,),
        in_specs=[
            pl.BlockSpec((NBLK, 128), lambda i: (i, 0)),
            pl.BlockSpec((NBLK, 128), lambda i: (i, 0)),
            pl.BlockSpec((256, 128), lambda i: (0, 0)),
            pl.BlockSpec((1, 128), lambda i: (0, 0)),
        ],
        out_specs=pl.BlockSpec((64, 128), lambda i: (0, 0)),
        out_shape=jax.ShapeDtypeStruct((64, 128), jnp.float32),
        scratch_shapes=[pltpu.VMEM((64, 128), jnp.float32),
                        pltpu.VMEM((64, 128), jnp.float32),
                        pltpu.VMEM((64, 128), jnp.float32)],
    )(x, batchb, wr, br)


def _sc_gather(a, b, row, col):
    n = a.shape[0]
    e = row.shape[0]
    nchunks = e // CKG
    mesh = plsc.VectorSubcoreMesh(core_axis_name="c", subcore_axis_name="s")

    @functools.partial(
        pl.kernel, mesh=mesh,
        out_type=(jax.ShapeDtypeStruct((e, 128), jnp.float32),
                  jax.ShapeDtypeStruct((e, 128), jnp.float32)),
        scratch_types=[
            pltpu.VMEM((2, CKG), jnp.int32),
            pltpu.VMEM((2, CKG), jnp.int32),
            pltpu.VMEM((2, CKG, 128), jnp.float32),
            pltpu.VMEM((2, CKG, 128), jnp.float32),
            pltpu.SemaphoreType.DMA,
            pltpu.SemaphoreType.DMA,
            pltpu.SemaphoreType.DMA,
            pltpu.SemaphoreType.DMA,
            pltpu.SemaphoreType.DMA,
            pltpu.SemaphoreType.DMA,
        ],
    )
    def k(a_hbm, b_hbm, row_hbm, col_hbm, g1_hbm, g2_hbm,
          idxr, idxc, bufa, bufb, si0, si1, sg0, sg1, sw0, sw1):
        sid = lax.axis_index("s")
        wid = sid * NC + lax.axis_index("c")
        nmine = (nchunks - wid + NW - 1) // NW
        sems_i = (si0, si1)
        sems_g = (sg0, sg1)
        sems_w = (sw0, sw1)

        def cbase(j):
            return (wid + j * NW) * CKG

        def start_idx(j, par):
            pltpu.async_copy(row_hbm.at[pl.ds(cbase(j), CKG)],
                             idxr.at[par], sems_i[par])
            pltpu.async_copy(col_hbm.at[pl.ds(cbase(j), CKG)],
                             idxc.at[par], sems_i[par])

        def wait_idx(par):
            pltpu.make_async_copy(row_hbm.at[pl.ds(0, CKG)],
                                  idxr.at[par], sems_i[par]).wait()
            pltpu.make_async_copy(col_hbm.at[pl.ds(0, CKG)],
                                  idxc.at[par], sems_i[par]).wait()

        def start_gather(par):
            pltpu.async_copy(a_hbm.at[idxr.at[par]], bufa.at[par],
                             sems_g[par])
            pltpu.async_copy(b_hbm.at[idxc.at[par]], bufb.at[par],
                             sems_g[par])

        def wait_gather(par):
            pltpu.make_async_copy(a_hbm.at[pl.ds(0, CKG)], bufa.at[par],
                                  sems_g[par]).wait()
            pltpu.make_async_copy(b_hbm.at[pl.ds(0, CKG)], bufb.at[par],
                                  sems_g[par]).wait()

        def start_write(j, par):
            pltpu.async_copy(bufa.at[par], g1_hbm.at[pl.ds(cbase(j), CKG)],
                             sems_w[par])
            pltpu.async_copy(bufb.at[par], g2_hbm.at[pl.ds(cbase(j), CKG)],
                             sems_w[par])

        def wait_write(par):
            pltpu.make_async_copy(g1_hbm.at[pl.ds(0, CKG)], bufa.at[par],
                                  sems_w[par]).wait()
            pltpu.make_async_copy(g2_hbm.at[pl.ds(0, CKG)], bufb.at[par],
                                  sems_w[par]).wait()

        start_idx(0, 0)

        def pair(t, _):
            j0 = 2 * t
            j1 = j0 + 1

            @pl.when(j1 < nmine)
            def _():
                start_idx(j1, 1)
            wait_idx(0)

            @pl.when(j0 >= 2)
            def _():
                wait_write(0)
            start_gather(0)

            @pl.when(j0 + 2 < nmine)
            def _():
                start_idx(j0 + 2, 0)

            @pl.when(j1 < nmine)
            def _():
                wait_idx(1)

                @pl.when(j1 >= 2)
                def _():
                    wait_write(1)
                start_gather(1)
            wait_gather(0)
            start_write(j0, 0)

            @pl.when(j1 < nmine)
            def _():
                wait_gather(1)
                start_write(j1, 1)
            return 0

        lax.fori_loop(0, (nmine + 1) // 2, pair, 0)
        wait_write(0)

        @pl.when(nmine >= 2)
        def _():
            wait_write(1)

    return k(a, b, row, col)


def _sc_scatter(p, w1, col, bnds, z128, z1):
    npad = z128.shape[0] * 8
    e = col.shape[0]
    ncnk = npad // 8
    nchunks = e // CKS
    rpt = ncnk // NS
    spt = npad // NS
    mesh = plsc.VectorSubcoreMesh(core_axis_name="c", subcore_axis_name="s")

    @functools.partial(
        pl.kernel, mesh=mesh,
        out_type=(jax.ShapeDtypeStruct((npad, 128), jnp.float32),
                  jax.ShapeDtypeStruct((2, npad), jnp.float32)),
        scratch_types=[
            pltpu.VMEM_SHARED((ncnk + 8, 128), jnp.float32),
            pltpu.VMEM_SHARED((npad,), jnp.float32),
            pltpu.VMEM((16,), jnp.int32),
            pltpu.VMEM((CKS,), jnp.int32),
            pltpu.VMEM((CKS,), jnp.int32),
            pltpu.VMEM((CKS, 128), jnp.float32),
            pltpu.VMEM((CKS,), jnp.float32),
        ],
    )
    def k(p_hbm, w_hbm, col_hbm, bnds_hbm, z128_hbm, z1_hbm, s_hbm, sp_hbm,
          acc, sacc, bbuf, ibuf, libuf, pbuf, wbuf):
        cid = lax.axis_index("c")
        sid = lax.axis_index("s")
        wid = sid * NC + cid
        pltpu.sync_copy(bnds_hbm, bbuf)
        bv = bbuf[pl.ds(0, 16)]

        for pp in range(4):
            nbase = (cid * 4 + pp) * ncnk
            lo = jnp.where(cid == 0, bv[2 * pp], bv[8 + 2 * pp])
            hi = jnp.where(cid == 0, bv[2 * pp + 1], bv[9 + 2 * pp])
            pltpu.sync_copy(z128_hbm.at[pl.ds(sid * rpt, rpt)],
                            acc.at[pl.ds(sid * rpt, rpt)])

            @pl.when(sid == 0)
            def _zdump():
                pltpu.sync_copy(z128_hbm.at[pl.ds(0, 8)],
                                acc.at[pl.ds(ncnk, 8)])

            plsc.subcore_barrier()

            def body(i, _):
                base = (lo + sid + i * NS) * CKS
                pltpu.sync_copy(col_hbm.at[pl.ds(base, CKS)], ibuf)
                pltpu.sync_copy(p_hbm.at[pl.ds(base, CKS)], pbuf)
                for kk in range(CKS // 16):
                    v = ibuf[pl.ds(kk * 16, 16)] - nbase
                    ok = (v >= 0) & (v < ncnk)
                    libuf[pl.ds(kk * 16, 16)] = jnp.where(ok, v, ncnk)
                pltpu.sync_copy(pbuf, acc.at[libuf], add=True)
                return 0

            nmine = jnp.maximum(0, (hi - lo - sid + NS - 1) // NS)
            lax.fori_loop(0, nmine, body, 0)
            plsc.subcore_barrier()
            pltpu.sync_copy(acc.at[pl.ds(sid * rpt, rpt)],
                            s_hbm.at[pl.ds(nbase + sid * rpt, rpt)])
            plsc.subcore_barrier()

        pltpu.sync_copy(z1_hbm.at[pl.ds(sid * spt, spt)],
                        sacc.at[pl.ds(sid * spt, spt)])
        plsc.subcore_barrier()

        def sbody(i, _):
            base = (wid + i * NW) * CKS
            pltpu.sync_copy(col_hbm.at[pl.ds(base, CKS)], ibuf)
            pltpu.sync_copy(w_hbm.at[pl.ds(base, CKS)], wbuf)
            pltpu.sync_copy(wbuf, sacc.at[ibuf], add=True)
            return 0

        nmine2 = (nchunks - wid + NW - 1) // NW
        lax.fori_loop(0, nmine2, sbody, 0)
        plsc.subcore_barrier()
        pltpu.sync_copy(sacc.at[pl.ds(sid * spt, spt)],
                        sp_hbm.at[cid, pl.ds(sid * spt, spt)])

    return k(p, w1, col, bnds, z128, z1)


def kernel(node_features, edge_index, edge_features, batch, params):
    n, nd = node_features.shape
    e = edge_index.shape[1]
    p = params
    h = 128
    nlayers = p['gW1'].shape[0]

    perm = jnp.argsort(edge_index[1])
    row = edge_index[0][perm]
    col = edge_index[1][perm]

    nfp = jnp.pad(node_features, ((0, 0), (0, h - nd)))
    wnp = jnp.pad(p['Wn'], ((0, h - nd), (0, 0)))
    efp = jnp.pad(edge_features, ((0, 0), (0, 16 - edge_features.shape[1])))[perm]

    w1r = p['gW1'][:, :h, :]
    w1c = p['gW1'][:, h:2 * h, :]
    w1e = p['gW1'][:, 2 * h:, :]
    m_l = jnp.pad(jnp.einsum('eh,lhk->lek', p['We'], w1e),
                  ((0, 0), (0, 6), (0, 0)))
    cb_l = (jnp.einsum('h,lhk->lk', p['be'], w1e)
            + p['gb1']).reshape(nlayers, 1, h)
    v_l = jnp.einsum('lhk,lko->lho', p['gW2'], p['gWa'])[..., 0] \
        .reshape(nlayers, 1, h)
    b2 = p['gb2'].reshape(nlayers, 1, h)
    wu1a = p['gWu1'][:, :h, :]
    wu1b = p['gWu1'][:, h:, :]
    bu1 = p['gbu1'].reshape(nlayers, 1, h)
    bu2 = p['gbu2'].reshape(nlayers, 1, h)
    br = p['br'].reshape(1, h)

    npad = 51200
    z128 = jnp.zeros((npad // 8, 128), jnp.float32)
    z1 = jnp.zeros((npad,), jnp.float32)
    batchb = jnp.broadcast_to(batch[:, None], (n, 128))

    ncnk = npad // 8
    edges_lo = jnp.searchsorted(col, jnp.arange(8) * ncnk).astype(jnp.int32)
    edges_hi = jnp.searchsorted(col, (jnp.arange(8) + 1) * ncnk).astype(jnp.int32)
    bnds = jnp.stack([edges_lo // CKS,
                      (edges_hi + CKS - 1) // CKS], axis=1).reshape(16)

    x = _tc_linear(nfp, wnp, p['bn'].reshape(1, h))

    for i in range(nlayers):
        a, b = _tc_ab(x, w1r[i], w1c[i])
        g1, g2 = _sc_gather(a, b, row, col)
        r, lg = _tc_edge(g1, g2, efp, m_l[i], cb_l[i], v_l[i])
        stats = _tc_softmax_stats(lg)
        pm, w3 = _tc_pscale(r, lg, stats)
        w1 = w3.reshape(e)
        s, spart = _sc_scatter(pm, w1, col, bnds, z128, z1)
        spart3 = spart.reshape(2, npad // NBLK, NBLK).transpose(1, 0, 2)
        x = _tc_update(x, s, spart3, p['gW2'][i], b2[i],
                       wu1a[i], wu1b[i], bu1[i], p['gWu2'][i], bu2[i])

    return _tc_readout(x, batchb, p['Wr'].reshape(256, h), br)


# speedup vs baseline: 1.4749x; 1.2613x over previous
import functools

import jax
import jax.numpy as jnp
from jax import lax
from jax.experimental import pallas as pl
from jax.experimental.pallas import tpu as pltpu
from jax.experimental.pallas import tpu_sc as plsc

NC = 2
NS = 16
NW = NC * NS
CKS = 64
CKG = 64

NBLK = 400
EBLK = 2000


def _linear_kernel(x_ref, w_ref, b_ref, o_ref):
    o_ref[...] = jnp.dot(x_ref[...], w_ref[...],
                         preferred_element_type=jnp.float32) + b_ref[...]


def _tc_linear(x, w, b, blk=NBLK):
    n, k = x.shape
    m = w.shape[1]
    return pl.pallas_call(
        _linear_kernel,
        grid=(n // blk,),
        in_specs=[
            pl.BlockSpec((blk, k), lambda i: (i, 0)),
            pl.BlockSpec((k, m), lambda i: (0, 0)),
            pl.BlockSpec((1, m), lambda i: (0, 0)),
        ],
        out_specs=pl.BlockSpec((blk, m), lambda i: (i, 0)),
        out_shape=jax.ShapeDtypeStruct((n, m), jnp.float32),
    )(x, w, b)


def _ab_kernel(x_ref, wr_ref, wc_ref, a_ref, b_ref):
    x = x_ref[...]
    a_ref[...] = jnp.dot(x, wr_ref[...], preferred_element_type=jnp.float32)
    b_ref[...] = jnp.dot(x, wc_ref[...], preferred_element_type=jnp.float32)


def _tc_ab(x, wr, wc):
    n, h = x.shape
    return pl.pallas_call(
        _ab_kernel,
        grid=(n // NBLK,),
        in_specs=[
            pl.BlockSpec((NBLK, h), lambda i: (i, 0)),
            pl.BlockSpec((h, h), lambda i: (0, 0)),
            pl.BlockSpec((h, h), lambda i: (0, 0)),
        ],
        out_specs=[
            pl.BlockSpec((NBLK, h), lambda i: (i, 0)),
            pl.BlockSpec((NBLK, h), lambda i: (i, 0)),
        ],
        out_shape=[
            jax.ShapeDtypeStruct((n, h), jnp.float32),
            jax.ShapeDtypeStruct((n, h), jnp.float32),
        ],
    )(x, wr, wc)


def _edge_kernel(g1_ref, g2_ref, ef_ref, m_ref, cb_ref, v_ref, r_ref, lg_ref):
    pre = (g1_ref[...] + g2_ref[...]
           + jnp.dot(ef_ref[...], m_ref[...],
                     preferred_element_type=jnp.float32)
           + cb_ref[...])
    r = jnp.maximum(pre, 0.0)
    r_ref[...] = r
    logit = lax.dot_general(v_ref[...], r, (((1,), (1,)), ((), ())),
                            preferred_element_type=jnp.float32)
    lg_ref[...] = logit.reshape(1, 1, EBLK)


def _tc_edge(g1, g2, efp, m, cb, v):
    e = g1.shape[0]
    nb = e // EBLK
    return pl.pallas_call(
        _edge_kernel,
        grid=(nb,),
        in_specs=[
            pl.BlockSpec((EBLK, 128), lambda i: (i, 0)),
            pl.BlockSpec((EBLK, 128), lambda i: (i, 0)),
            pl.BlockSpec((EBLK, 16), lambda i: (i, 0)),
            pl.BlockSpec((16, 128), lambda i: (0, 0)),
            pl.BlockSpec((1, 128), lambda i: (0, 0)),
            pl.BlockSpec((1, 128), lambda i: (0, 0)),
        ],
        out_specs=[
            pl.BlockSpec((EBLK, 128), lambda i: (i, 0)),
            pl.BlockSpec((1, 1, EBLK), lambda i: (i, 0, 0)),
        ],
        out_shape=[
            jax.ShapeDtypeStruct((e, 128), jnp.float32),
            jax.ShapeDtypeStruct((nb, 1, EBLK), jnp.float32),
        ],
    )(g1, g2, efp, m, cb, v)


def _softmax_stats_kernel(lg_ref, o_ref, m_ref, z_ref):
    i = pl.program_id(0)

    @pl.when(i == 0)
    def _init():
        m_ref[0, 0] = -jnp.inf
        z_ref[0, 0] = 0.0

    blk = lg_ref[0, 0, :]
    bm = jnp.max(blk)
    m_old = m_ref[0, 0]
    m_new = jnp.maximum(m_old, bm)
    z_new = (z_ref[0, 0] * jnp.exp(m_old - m_new)
             + jnp.sum(jnp.exp(blk - m_new)))
    m_ref[0, 0] = m_new
    z_ref[0, 0] = z_new
    lane = lax.broadcasted_iota(jnp.int32, (1, 128), 1)
    o_ref[...] = jnp.where(lane == 0, m_new,
                           jnp.where(lane == 1, z_new, 0.0))


def _tc_softmax_stats(lg):
    nb = lg.shape[0]
    return pl.pallas_call(
        _softmax_stats_kernel,
        grid=(nb,),
        in_specs=[pl.BlockSpec((1, 1, EBLK), lambda i: (i, 0, 0))],
        out_specs=pl.BlockSpec((1, 128), lambda i: (0, 0)),
        out_shape=jax.ShapeDtypeStruct((1, 128), jnp.float32),
        scratch_shapes=[pltpu.SMEM((1, 1), jnp.float32),
                        pltpu.SMEM((1, 1), jnp.float32)],
    )(lg)


def _pscale_kernel(r_ref, lg_ref, st_ref, p_ref, w_ref):
    m = st_ref[0, 0]
    z = st_ref[0, 1]
    w = jnp.exp(lg_ref[0] - m) * (1.0 / z)
    wmat = lax.dot_general(w, jnp.ones((1, 128), jnp.float32),
                           (((0,), (0,)), ((), ())),
                           preferred_element_type=jnp.float32)
    p_ref[...] = r_ref[...] * wmat
    w_ref[...] = w.reshape(1, 1, EBLK)


def _tc_pscale(r, lg, stats):
    e = r.shape[0]
    nb = e // EBLK
    return pl.pallas_call(
        _pscale_kernel,
        grid=(nb,),
        in_specs=[
            pl.BlockSpec((EBLK, 128), lambda i: (i, 0)),
            pl.BlockSpec((1, 1, EBLK), lambda i: (i, 0, 0)),
            pl.BlockSpec((1, 128), lambda i: (0, 0)),
        ],
        out_specs=[
            pl.BlockSpec((EBLK, 128), lambda i: (i, 0)),
            pl.BlockSpec((1, 1, EBLK), lambda i: (i, 0, 0)),
        ],
        out_shape=[
            jax.ShapeDtypeStruct((e, 128), jnp.float32),
            jax.ShapeDtypeStruct((nb, 1, EBLK), jnp.float32),
        ],
    )(r, lg, stats)


def _upd_kernel(x_ref, s_ref, sp_ref, w2_ref, b2_ref, wa_ref, wb_ref,
                bu1_ref, wu2_ref, bu2_ref, o_ref):
    x = x_ref[...]
    s2d = sp_ref[0, 0:1, :] + sp_ref[0, 1:2, :]
    smat = lax.dot_general(s2d, jnp.ones((1, 128), jnp.float32),
                           (((0,), (0,)), ((), ())),
                           preferred_element_type=jnp.float32)
    agg = jnp.dot(s_ref[...], w2_ref[...], preferred_element_type=jnp.float32)
    agg = agg + smat * b2_ref[...]
    h = jnp.maximum(
        jnp.dot(x, wa_ref[...], preferred_element_type=jnp.float32)
        + jnp.dot(agg, wb_ref[...], preferred_element_type=jnp.float32)
        + bu1_ref[...], 0.0)
    upd = jnp.dot(h, wu2_ref[...], preferred_element_type=jnp.float32) \
        + bu2_ref[...]
    o_ref[...] = x + upd


def _tc_update(x, s, spart3, w2, b2, wa, wb, bu1, wu2, bu2):
    n = x.shape[0]
    return pl.pallas_call(
        _upd_kernel,
        grid=(n // NBLK,),
        in_specs=[
            pl.BlockSpec((NBLK, 128), lambda i: (i, 0)),
            pl.BlockSpec((NBLK, 128), lambda i: (i, 0)),
            pl.BlockSpec((1, 2, NBLK), lambda i: (i, 0, 0)),
            pl.BlockSpec((128, 128), lambda i: (0, 0)),
            pl.BlockSpec((1, 128), lambda i: (0, 0)),
            pl.BlockSpec((128, 128), lambda i: (0, 0)),
            pl.BlockSpec((128, 128), lambda i: (0, 0)),
            pl.BlockSpec((1, 128), lambda i: (0, 0)),
            pl.BlockSpec((128, 128), lambda i: (0, 0)),
            pl.BlockSpec((1, 128), lambda i: (0, 0)),
        ],
        out_specs=pl.BlockSpec((NBLK, 128), lambda i: (i, 0)),
        out_shape=jax.ShapeDtypeStruct((n, 128), jnp.float32),
    )(x, s, spart3, w2, b2, wa, wb, bu1, wu2, bu2)


def _readout_kernel(x_ref, bt_ref, wr_ref, br_ref, o_ref,
                    sum_ref, max_ref, cnt_ref):
    i = pl.program_id(0)
    nb = pl.num_programs(0)

    @pl.when(i == 0)
    def _init():
        sum_ref[...] = jnp.zeros((64, 128), jnp.float32)
        max_ref[...] = jnp.full((64, 128), -jnp.inf, jnp.float32)
        cnt_ref[...] = jnp.zeros((64, 128), jnp.float32)

    x = x_ref[...]
    b = bt_ref[...]
    oh = (b[:, 0:64] == lax.broadcasted_iota(jnp.int32, (NBLK, 64), 1)
          ).astype(jnp.float32)
    sum_ref[...] += lax.dot_general(oh, x, (((0,), (0,)), ((), ())),
                                    preferred_element_type=jnp.float32)
    cnt_ref[...] += lax.dot_general(oh, jnp.ones((NBLK, 128), jnp.float32),
                                    (((0,), (0,)), ((), ())),
                                    preferred_element_type=jnp.float32)

    blo = jnp.min(b)
    bhi = jnp.max(b)
    seg = lax.broadcasted_iota(jnp.int32, (64, 128), 0)

    def mbody(bb, _):
        mb = jnp.max(jnp.where(b == bb, x, -jnp.inf), axis=0, keepdims=True)
        max_ref[...] = jnp.maximum(
            max_ref[...], jnp.where(seg == bb, mb, -jnp.inf))
        return 0

    lax.fori_loop(blo, bhi + 1, mbody, 0)

    @pl.when(i == nb - 1)
    def _final():
        mean = sum_ref[...] / jnp.maximum(cnt_ref[...], 1.0)
        g = jnp.concatenate([mean, max_ref[...]], axis=1)
        o_ref[...] = jnp.maximum(
            jnp.dot(g, wr_ref[...], preferred_element_type=jnp.float32)
            + br_ref[...], 0.0)


def _tc_readout(x, batchb, wr, br):
    n = x.shape[0]
    return pl.pallas_call(
        _readout_kernel,
        grid=(n // NBLK,),
        in_specs=[
            pl.BlockSpec((NBLK, 128), lambda i: (i, 0)),
            pl.BlockSpec((NBLK, 128), lambda i: (i, 0)),
            pl.BlockSpec((256, 128), lambda i: (0, 0)),
            pl.BlockSpec((1, 128), lambda i: (0, 0)),
        ],
        out_specs=pl.BlockSpec((64, 128), lambda i: (0, 0)),
        out_shape=jax.ShapeDtypeStruct((64, 128), jnp.float32),
        scratch_shapes=[pltpu.VMEM((64, 128), jnp.float32),
                        pltpu.VMEM((64, 128), jnp.float32),
                        pltpu.VMEM((64, 128), jnp.float32)],
    )(x, batchb, wr, br)


def _sc_gather(a, b, row, col):
    n = a.shape[0]
    e = row.shape[0]
    nchunks = e // CKG
    mesh = plsc.VectorSubcoreMesh(core_axis_name="c", subcore_axis_name="s")

    @functools.partial(
        pl.kernel, mesh=mesh,
        out_type=(jax.ShapeDtypeStruct((e, 128), jnp.float32),
                  jax.ShapeDtypeStruct((e, 128), jnp.float32)),
        scratch_types=[
            pltpu.VMEM((2, CKG), jnp.int32),
            pltpu.VMEM((2, CKG), jnp.int32),
            pltpu.VMEM((2, CKG, 128), jnp.float32),
            pltpu.VMEM((2, CKG, 128), jnp.float32),
            pltpu.SemaphoreType.DMA,
            pltpu.SemaphoreType.DMA,
            pltpu.SemaphoreType.DMA,
            pltpu.SemaphoreType.DMA,
            pltpu.SemaphoreType.DMA,
            pltpu.SemaphoreType.DMA,
        ],
    )
    def k(a_hbm, b_hbm, row_hbm, col_hbm, g1_hbm, g2_hbm,
          idxr, idxc, bufa, bufb, si0, si1, sg0, sg1, sw0, sw1):
        sid = lax.axis_index("s")
        wid = sid * NC + lax.axis_index("c")
        nmine = (nchunks - wid + NW - 1) // NW
        sems_i = (si0, si1)
        sems_g = (sg0, sg1)
        sems_w = (sw0, sw1)

        def cbase(j):
            return (wid + j * NW) * CKG

        def start_idx(j, par):
            pltpu.async_copy(row_hbm.at[pl.ds(cbase(j), CKG)],
                             idxr.at[par], sems_i[par])
            pltpu.async_copy(col_hbm.at[pl.ds(cbase(j), CKG)],
                             idxc.at[par], sems_i[par])

        def wait_idx(par):
            pltpu.make_async_copy(row_hbm.at[pl.ds(0, CKG)],
                                  idxr.at[par], sems_i[par]).wait()
            pltpu.make_async_copy(col_hbm.at[pl.ds(0, CKG)],
                                  idxc.at[par], sems_i[par]).wait()

        def start_gather(par):
            pltpu.async_copy(a_hbm.at[idxr.at[par]], bufa.at[par],
                             sems_g[par])
            pltpu.async_copy(b_hbm.at[idxc.at[par]], bufb.at[par],
                             sems_g[par])

        def wait_gather(par):
            pltpu.make_async_copy(a_hbm.at[pl.ds(0, CKG)], bufa.at[par],
                                  sems_g[par]).wait()
            pltpu.make_async_copy(b_hbm.at[pl.ds(0, CKG)], bufb.at[par],
                                  sems_g[par]).wait()

        def start_write(j, par):
            pltpu.async_copy(bufa.at[par], g1_hbm.at[pl.ds(cbase(j), CKG)],
                             sems_w[par])
            pltpu.async_copy(bufb.at[par], g2_hbm.at[pl.ds(cbase(j), CKG)],
                             sems_w[par])

        def wait_write(par):
            pltpu.make_async_copy(g1_hbm.at[pl.ds(0, CKG)], bufa.at[par],
                                  sems_w[par]).wait()
            pltpu.make_async_copy(g2_hbm.at[pl.ds(0, CKG)], bufb.at[par],
                                  sems_w[par]).wait()

        start_idx(0, 0)

        def pair(t, _):
            j0 = 2 * t
            j1 = j0 + 1

            @pl.when(j1 < nmine)
            def _():
                start_idx(j1, 1)
            wait_idx(0)

            @pl.when(j0 >= 2)
            def _():
                wait_write(0)
            start_gather(0)

            @pl.when(j0 + 2 < nmine)
            def _():
                start_idx(j0 + 2, 0)

            @pl.when(j1 < nmine)
            def _():
                wait_idx(1)

                @pl.when(j1 >= 2)
                def _():
                    wait_write(1)
                start_gather(1)
            wait_gather(0)
            start_write(j0, 0)

            @pl.when(j1 < nmine)
            def _():
                wait_gather(1)
                start_write(j1, 1)
            return 0

        lax.fori_loop(0, (nmine + 1) // 2, pair, 0)
        wait_write(0)

        @pl.when(nmine >= 2)
        def _():
            wait_write(1)

    return k(a, b, row, col)


def _sc_scatter(p, w1, col, bnds, z128, z1):
    npad = z128.shape[0] * 8
    e = col.shape[0]
    ncnk = npad // 8
    nchunks = e // CKS
    rpt = ncnk // NS
    spt = npad // NS
    mesh = plsc.VectorSubcoreMesh(core_axis_name="c", subcore_axis_name="s")

    @functools.partial(
        pl.kernel, mesh=mesh,
        out_type=(jax.ShapeDtypeStruct((npad, 128), jnp.float32),
                  jax.ShapeDtypeStruct((2, npad), jnp.float32)),
        scratch_types=[
            pltpu.VMEM_SHARED((ncnk + 8, 128), jnp.float32),
            pltpu.VMEM_SHARED((npad,), jnp.float32),
            pltpu.VMEM((16,), jnp.int32),
            pltpu.VMEM((CKS,), jnp.int32),
            pltpu.VMEM((CKS,), jnp.int32),
            pltpu.VMEM((CKS, 128), jnp.float32),
            pltpu.VMEM((CKS,), jnp.float32),
        ],
    )
    def k(p_hbm, w_hbm, col_hbm, bnds_hbm, z128_hbm, z1_hbm, s_hbm, sp_hbm,
          acc, sacc, bbuf, ibuf, libuf, pbuf, wbuf):
        cid = lax.axis_index("c")
        sid = lax.axis_index("s")
        wid = sid * NC + cid
        pltpu.sync_copy(bnds_hbm, bbuf)
        bv = bbuf[pl.ds(0, 16)]

        for pp in range(4):
            nbase = (cid * 4 + pp) * ncnk
            lo = jnp.where(cid == 0, bv[2 * pp], bv[8 + 2 * pp])
            hi = jnp.where(cid == 0, bv[2 * pp + 1], bv[9 + 2 * pp])
            pltpu.sync_copy(z128_hbm.at[pl.ds(sid * rpt, rpt)],
                            acc.at[pl.ds(sid * rpt, rpt)])

            @pl.when(sid == 0)
            def _zdump():
                pltpu.sync_copy(z128_hbm.at[pl.ds(0, 8)],
                                acc.at[pl.ds(ncnk, 8)])

            plsc.subcore_barrier()

            def body(i, _):
                base = (lo + sid + i * NS) * CKS
                pltpu.sync_copy(col_hbm.at[pl.ds(base, CKS)], ibuf)
                pltpu.sync_copy(p_hbm.at[pl.ds(base, CKS)], pbuf)
                for kk in range(CKS // 16):
                    v = ibuf[pl.ds(kk * 16, 16)] - nbase
                    ok = (v >= 0) & (v < ncnk)
                    libuf[pl.ds(kk * 16, 16)] = jnp.where(ok, v, ncnk)
                pltpu.sync_copy(pbuf, acc.at[libuf], add=True)
                return 0

            nmine = jnp.maximum(0, (hi - lo - sid + NS - 1) // NS)
            lax.fori_loop(0, nmine, body, 0)
            plsc.subcore_barrier()
            pltpu.sync_copy(acc.at[pl.ds(sid * rpt, rpt)],
                            s_hbm.at[pl.ds(nbase + sid * rpt, rpt)])
            plsc.subcore_barrier()

        pltpu.sync_copy(z1_hbm.at[pl.ds(sid * spt, spt)],
                        sacc.at[pl.ds(sid * spt, spt)])
        plsc.subcore_barrier()

        def sbody(i, _):
            base = (wid + i * NW) * CKS
            pltpu.sync_copy(col_hbm.at[pl.ds(base, CKS)], ibuf)
            pltpu.sync_copy(w_hbm.at[pl.ds(base, CKS)], wbuf)
            pltpu.sync_copy(wbuf, sacc.at[ibuf], add=True)
            return 0

        nmine2 = (nchunks - wid + NW - 1) // NW
        lax.fori_loop(0, nmine2, sbody, 0)
        plsc.subcore_barrier()
        pltpu.sync_copy(sacc.at[pl.ds(sid * spt, spt)],
                        sp_hbm.at[cid, pl.ds(sid * spt, spt)])

    return k(p, w1, col, bnds, z128, z1)


def kernel(node_features, edge_index, edge_features, batch, params):
    n, nd = node_features.shape
    e = edge_index.shape[1]
    p = params
    h = 128
    nlayers = p['gW1'].shape[0]

    perm = jnp.argsort(edge_index[1])
    row = edge_index[0][perm]
    col = edge_index[1][perm]

    nfp = jnp.pad(node_features, ((0, 0), (0, h - nd)))
    wnp = jnp.pad(p['Wn'], ((0, h - nd), (0, 0)))
    efp = jnp.pad(edge_features, ((0, 0), (0, 16 - edge_features.shape[1])))[perm]

    w1r = p['gW1'][:, :h, :]
    w1c = p['gW1'][:, h:2 * h, :]
    w1e = p['gW1'][:, 2 * h:, :]
    m_l = jnp.pad(jnp.einsum('eh,lhk->lek', p['We'], w1e),
                  ((0, 0), (0, 6), (0, 0)))
    cb_l = (jnp.einsum('h,lhk->lk', p['be'], w1e)
            + p['gb1']).reshape(nlayers, 1, h)
    v_l = jnp.einsum('lhk,lko->lho', p['gW2'], p['gWa'])[..., 0] \
        .reshape(nlayers, 1, h)
    b2 = p['gb2'].reshape(nlayers, 1, h)
    wu1a = p['gWu1'][:, :h, :]
    wu1b = p['gWu1'][:, h:, :]
    bu1 = p['gbu1'].reshape(nlayers, 1, h)
    bu2 = p['gbu2'].reshape(nlayers, 1, h)
    br = p['br'].reshape(1, h)

    npad = 51200
    z128 = jnp.zeros((npad // 8, 128), jnp.float32)
    z1 = jnp.zeros((npad,), jnp.float32)
    batchb = jnp.broadcast_to(batch[:, None], (n, 128))

    ncnk = npad // 8
    edges_lo = jnp.searchsorted(col, jnp.arange(8) * ncnk).astype(jnp.int32)
    edges_hi = jnp.searchsorted(col, (jnp.arange(8) + 1) * ncnk).astype(jnp.int32)
    bnds = jnp.stack([edges_lo // CKS,
                      (edges_hi + CKS - 1) // CKS], axis=1).reshape(16)

    x = _tc_linear(nfp, wnp, p['bn'].reshape(1, h))

    for i in range(nlayers):
        a, b = _tc_ab(x, w1r[i], w1c[i])
        g1, g2 = _sc_gather(a, b, row, col)
        r, lg = _tc_edge(g1, g2, efp, m_l[i], cb_l[i], v_l[i])
        stats = _tc_softmax_stats(lg)
        pm, w3 = _tc_pscale(r, lg, stats)
        w1 = w3.reshape(e)
        s, spart = _sc_scatter(pm, w1, col, bnds, z128, z1)
        spart3 = spart.reshape(2, npad // NBLK, NBLK).transpose(1, 0, 2)
        x = _tc_update(x, s, spart3, p['gW2'][i], b2[i],
                       wu1a[i], wu1b[i], bu1[i], p['gWu2'][i], bu2[i])

    return _tc_readout(x, batchb, p['Wr'].reshape(256, h), br)
